# baseline (speedup 1.0000x reference)
import jax
import jax.numpy as jnp
from jax import lax
from jax.experimental import pallas as pl
from jax.experimental.pallas import tpu as pltpu
from jax.experimental.pallas import tpu_sc as plsc

VOCAB = 1000000
EMBED_DIM = 64
BATCH = 16384

NUM_CORES = 2
NUM_SUBCORES = 16
LANES = 16

NUM_WORKERS = NUM_CORES * NUM_SUBCORES
ROWS_PER_WORKER = 288
BATCH_SC = NUM_WORKERS * ROWS_PER_WORKER
NUM_SLOTS = 7
IDX_PAD = ROWS_PER_WORKER + LANES


BLOCK = 128


def _fetch_row(tab_hbm, blocks_v, idx_v, j, slot, sems, table_id):
    vq = idx_v[pl.ds(j, LANES)]
    v = vq[0]
    off = pl.multiple_of((v // BLOCK) * BLOCK, BLOCK)
    pltpu.async_copy(tab_hbm.at[:, pl.ds(off, BLOCK)],
                     blocks_v.at[:, pl.ds(slot * BLOCK, BLOCK)],
                     sems.at[table_id * NUM_SLOTS + slot])


def _sc_body(cw_hbm, xw_hbm, ctab_hbm, xtab_hbm, out_hbm,
             idxc_v, idxx_v, blkc_v, blkx_v, out_v, sems):
    wid = lax.axis_index("s") * NUM_CORES + lax.axis_index("c")
    base = wid * ROWS_PER_WORKER

    pltpu.sync_copy(cw_hbm.at[pl.ds(base, ROWS_PER_WORKER)],
                    idxc_v.at[pl.ds(0, ROWS_PER_WORKER)])
    pltpu.sync_copy(xw_hbm.at[pl.ds(base, ROWS_PER_WORKER)],
                    idxx_v.at[pl.ds(0, ROWS_PER_WORKER)])

    lane_iota = lax.iota(jnp.int32, LANES)
    last_mask = lane_iota == (LANES - 1)

    for s in range(NUM_SLOTS):
        _fetch_row(ctab_hbm, blkc_v, idxc_v, s, s, sems, 0)
        _fetch_row(xtab_hbm, blkx_v, idxx_v, s, s, sems, 1)

    def step(j, s):
        slot_off = s * BLOCK
        vc = idxc_v[pl.ds(j, LANES)]
        vx = idxx_v[pl.ds(j, LANES)]
        rc = vc[0] % BLOCK
        rx = vx[0] % BLOCK

        pltpu.make_async_copy(ctab_hbm.at[:, pl.ds(0, BLOCK)],
                              blkc_v.at[:, pl.ds(slot_off, BLOCK)],
                              sems.at[s]).wait()
        pltpu.make_async_copy(xtab_hbm.at[:, pl.ds(0, BLOCK)],
                              blkx_v.at[:, pl.ds(slot_off, BLOCK)],
                              sems.at[NUM_SLOTS + s]).wait()

        colc = jnp.full((LANES,), slot_off, jnp.int32) + rc
        colx = jnp.full((LANES,), slot_off, jnp.int32) + rx
        acc = jnp.zeros((LANES,), jnp.float32)
        for k in range(EMBED_DIM // LANES):
            rows = lane_iota + k * LANES
            c = plsc.load_gather(blkc_v, [rows, colc])
            x = plsc.load_gather(blkx_v, [rows, colx])
            acc = acc + c * x
        total = plsc.cumsum(acc)
        plsc.store_scatter(out_v, [jnp.full((LANES,), j, jnp.int32)],
                           total, mask=last_mask)

        @pl.when(j + NUM_SLOTS < ROWS_PER_WORKER)
        def _():
            _fetch_row(ctab_hbm, blkc_v, idxc_v, j + NUM_SLOTS, s, sems, 0)
            _fetch_row(xtab_hbm, blkx_v, idxx_v, j + NUM_SLOTS, s, sems, 1)

    def group(g, carry):
        for s in range(NUM_SLOTS):
            step(g * NUM_SLOTS + s, s)
        return carry

    num_groups = ROWS_PER_WORKER // NUM_SLOTS
    lax.fori_loop(0, num_groups, group, 0)
    for s in range(ROWS_PER_WORKER % NUM_SLOTS):
        step(num_groups * NUM_SLOTS + s, s)

    def sig_body(chunk, carry):
        sl = pl.ds(chunk * LANES, LANES)
        out_v[sl] = 1.0 / (1.0 + jnp.exp(-out_v[sl]))
        return carry

    lax.fori_loop(0, ROWS_PER_WORKER // LANES, sig_body, 0)

    pltpu.sync_copy(out_v, out_hbm.at[pl.ds(base, ROWS_PER_WORKER)])


def kernel(center_words, context_words, center_table, context_table):
    mesh = plsc.VectorSubcoreMesh(core_axis_name="c", subcore_axis_name="s")
    run = pl.kernel(
        _sc_body,
        out_type=jax.ShapeDtypeStruct((BATCH_SC,), jnp.float32),
        mesh=mesh,
        scratch_types=[
            pltpu.VMEM((IDX_PAD,), jnp.int32),
            pltpu.VMEM((IDX_PAD,), jnp.int32),
            pltpu.VMEM((EMBED_DIM, NUM_SLOTS * BLOCK), jnp.float32),
            pltpu.VMEM((EMBED_DIM, NUM_SLOTS * BLOCK), jnp.float32),
            pltpu.VMEM((ROWS_PER_WORKER,), jnp.float32),
            pltpu.SemaphoreType.DMA((2 * NUM_SLOTS,)),
        ],
        compiler_params=pltpu.CompilerParams(
            needs_layout_passes=False, use_tc_tiling_on_sc=True),
    )
    cw = center_words.astype(jnp.int32)
    xw = context_words.astype(jnp.int32)
    out_sc = run(cw[:BATCH_SC], xw[:BATCH_SC],
                 center_table.T, context_table.T)
    ce = jnp.take(center_table, cw[BATCH_SC:], axis=0)
    xe = jnp.take(context_table, xw[BATCH_SC:], axis=0)
    dot = jnp.sum(ce * xe, axis=1)
    out_tc = 1.0 / (1.0 + jnp.exp(-dot))
    return jnp.concatenate([out_sc, out_tc])


# speedup vs baseline: 1.7705x; 1.7705x over previous
import jax
import jax.numpy as jnp
from jax import lax
from jax.experimental import pallas as pl
from jax.experimental.pallas import tpu as pltpu
from jax.experimental.pallas import tpu_sc as plsc

VOCAB = 1000000
EMBED_DIM = 64
BATCH = 16384

NUM_CORES = 2
NUM_SUBCORES = 16
LANES = 16

NUM_WORKERS = NUM_CORES * NUM_SUBCORES
ROWS_PER_WORKER = BATCH // NUM_WORKERS
NUM_SLOTS = 7
IDX_PAD = ROWS_PER_WORKER + LANES


BLOCK = 128


def _fetch_row(tab_hbm, blocks_v, idx_v, j, slot, sems, table_id):
    vq = idx_v[pl.ds(j, LANES)]
    v = vq[0]
    off = pl.multiple_of((v // BLOCK) * BLOCK, BLOCK)
    pltpu.async_copy(tab_hbm.at[:, pl.ds(off, BLOCK)],
                     blocks_v.at[:, pl.ds(slot * BLOCK, BLOCK)],
                     sems.at[table_id * NUM_SLOTS + slot])


def _sc_body(cw_hbm, xw_hbm, ctab_hbm, xtab_hbm, out_hbm,
             idxc_v, idxx_v, blkc_v, blkx_v, out_v, sems):
    wid = lax.axis_index("s") * NUM_CORES + lax.axis_index("c")
    base = wid * ROWS_PER_WORKER

    pltpu.sync_copy(cw_hbm.at[pl.ds(base, ROWS_PER_WORKER)],
                    idxc_v.at[pl.ds(0, ROWS_PER_WORKER)])
    pltpu.sync_copy(xw_hbm.at[pl.ds(base, ROWS_PER_WORKER)],
                    idxx_v.at[pl.ds(0, ROWS_PER_WORKER)])

    lane_iota = lax.iota(jnp.int32, LANES)
    last_mask = lane_iota == (LANES - 1)

    for s in range(NUM_SLOTS):
        _fetch_row(ctab_hbm, blkc_v, idxc_v, s, s, sems, 0)
        _fetch_row(xtab_hbm, blkx_v, idxx_v, s, s, sems, 1)

    def step(j, s):
        slot_off = s * BLOCK
        vc = idxc_v[pl.ds(j, LANES)]
        vx = idxx_v[pl.ds(j, LANES)]
        rc = vc[0] % BLOCK
        rx = vx[0] % BLOCK

        pltpu.make_async_copy(ctab_hbm.at[:, pl.ds(0, BLOCK)],
                              blkc_v.at[:, pl.ds(slot_off, BLOCK)],
                              sems.at[s]).wait()
        pltpu.make_async_copy(xtab_hbm.at[:, pl.ds(0, BLOCK)],
                              blkx_v.at[:, pl.ds(slot_off, BLOCK)],
                              sems.at[NUM_SLOTS + s]).wait()

        colc = jnp.full((LANES,), slot_off, jnp.int32) + rc
        colx = jnp.full((LANES,), slot_off, jnp.int32) + rx
        acc = jnp.zeros((LANES,), jnp.float32)
        for k in range(EMBED_DIM // LANES):
            rows = lane_iota + k * LANES
            c = plsc.load_gather(blkc_v, [rows, colc])
            x = plsc.load_gather(blkx_v, [rows, colx])
            acc = acc + c * x
        total = plsc.cumsum(acc)
        plsc.store_scatter(out_v, [jnp.full((LANES,), j, jnp.int32)],
                           total, mask=last_mask)

        @pl.when(j + NUM_SLOTS < ROWS_PER_WORKER)
        def _():
            _fetch_row(ctab_hbm, blkc_v, idxc_v, j + NUM_SLOTS, s, sems, 0)
            _fetch_row(xtab_hbm, blkx_v, idxx_v, j + NUM_SLOTS, s, sems, 1)

    def group(g, carry):
        for s in range(NUM_SLOTS):
            step(g * NUM_SLOTS + s, s)
        return carry

    num_groups = ROWS_PER_WORKER // NUM_SLOTS
    lax.fori_loop(0, num_groups, group, 0)
    for s in range(ROWS_PER_WORKER % NUM_SLOTS):
        step(num_groups * NUM_SLOTS + s, s)

    def sig_body(chunk, carry):
        sl = pl.ds(chunk * LANES, LANES)
        out_v[sl] = 1.0 / (1.0 + jnp.exp(-out_v[sl]))
        return carry

    lax.fori_loop(0, ROWS_PER_WORKER // LANES, sig_body, 0)

    pltpu.sync_copy(out_v, out_hbm.at[pl.ds(base, ROWS_PER_WORKER)])


def kernel(center_words, context_words, center_table, context_table):
    mesh = plsc.VectorSubcoreMesh(core_axis_name="c", subcore_axis_name="s")
    run = pl.kernel(
        _sc_body,
        out_type=jax.ShapeDtypeStruct((BATCH,), jnp.float32),
        mesh=mesh,
        scratch_types=[
            pltpu.VMEM((IDX_PAD,), jnp.int32),
            pltpu.VMEM((IDX_PAD,), jnp.int32),
            pltpu.VMEM((EMBED_DIM, NUM_SLOTS * BLOCK), jnp.float32),
            pltpu.VMEM((EMBED_DIM, NUM_SLOTS * BLOCK), jnp.float32),
            pltpu.VMEM((ROWS_PER_WORKER,), jnp.float32),
            pltpu.SemaphoreType.DMA((2 * NUM_SLOTS,)),
        ],
        compiler_params=pltpu.CompilerParams(
            needs_layout_passes=False, use_tc_tiling_on_sc=True),
    )
    return run(center_words.astype(jnp.int32), context_words.astype(jnp.int32),
               center_table.T, context_table.T)
